# Initial kernel scaffold; baseline (speedup 1.0000x reference)
import jax
import jax.numpy as jnp
from jax.experimental import pallas as pl


def kernel(times, depths, motion):
    raise NotImplementedError("write your pallas kernel here")


# speedup vs baseline: 88.3563x; 88.3563x over previous
import functools

import jax
import jax.numpy as jnp
import numpy as np
from jax import lax
from jax.experimental import pallas as pl
from jax.experimental.pallas import tpu as pltpu
from jax.experimental.pallas import tpu_sc as plsc

BOUND = 0.1
BIN = 1e-4
KW = 1e-3
D = 16
EPS = 0.001
NUM_T = 10000
N = 2097152

NUM_CORES = 2
NUM_SUBCORES = 16
NW = NUM_CORES * NUM_SUBCORES
PER_W = N // NW
CHUNK = 8192
NPAIR = D // 2

DS = float(1.0 / (D - 1) + EPS)
EPS2 = float(EPS * EPS)
_R0 = float(1.0 / 0.06915)


def _conv_taps():
    kernel_slope = 0.5 * KW / BIN
    half_kernel = np.arange(1.0, 0.0, -1.0 / kernel_slope)
    k = np.concatenate([half_kernel[::-1], half_kernel[1:]])
    k /= np.sum(k)
    return k


_TAPS = _conv_taps()
_HALF = (len(_TAPS) - 1) // 2
_CONV_ONES = np.convolve(np.ones(NUM_T), _TAPS, mode="same").astype(np.float32)


def _tc_table_body(mp_ref, ones_ref, out_ref):
    m0 = BOUND * jnp.tanh(mp_ref[...])
    z = jnp.zeros((D, _HALF), jnp.float32)
    m = jnp.concatenate([z, m0, z], axis=1)
    acc = _TAPS[0] * m[:, 0:NUM_T]
    for k in range(1, len(_TAPS)):
        acc = acc + _TAPS[k] * m[:, k:k + NUM_T]
    sm = acc / ones_ref[...]
    sm = sm - jnp.mean(sm, axis=1, keepdims=True)
    lo = lax.bitcast_convert_type(sm[:NPAIR, :], jnp.uint32)
    hi = lax.bitcast_convert_type(sm[NPAIR:, :], jnp.uint32)
    lo = lo + jnp.uint32(0x7FFF) + ((lo >> 16) & jnp.uint32(1))
    hi = hi + jnp.uint32(0x7FFF) + ((hi >> 16) & jnp.uint32(1))
    packed = (hi & jnp.uint32(0xFFFF0000)) | (lo >> 16)
    out_ref[...] = lax.bitcast_convert_type(packed, jnp.int32)


def _build_table(motion):
    perm = jnp.array(
        [i for i in range(0, D, 2)] + [i for i in range(1, D, 2)], jnp.int32)
    ones = jnp.asarray(_CONV_ONES).reshape(1, NUM_T)
    return pl.pallas_call(
        _tc_table_body,
        out_shape=jax.ShapeDtypeStruct((NPAIR, NUM_T), jnp.int32),
    )(motion[perm], ones).reshape(-1)


NCH = PER_W // CHUNK


def _sc_body(table_hbm, times_hbm, depths_hbm, out_hbm, tab_v,
             ta_v, tb_v, da_v, db_v, oa_v, ob_v,
             sta, stb, sda, sdb, soa, sob):
    wid = lax.axis_index("s") * NUM_CORES + lax.axis_index("c")
    base = wid * PER_W
    t_bufs, d_bufs, o_bufs = (ta_v, tb_v), (da_v, db_v), (oa_v, ob_v)
    st, sd, so = (sta, stb), (sda, sdb), (soa, sob)

    pltpu.async_copy(times_hbm.at[pl.ds(base, CHUNK)], ta_v, sta)
    pltpu.async_copy(depths_hbm.at[pl.ds(base, CHUNK)], da_v, sda)
    pltpu.sync_copy(table_hbm, tab_v)

    def make_chunk_body(b):
      def chunk_body(ci, t_v, d_v, o_v):
        off = base + ci * CHUNK
        pltpu.make_async_copy(
            times_hbm.at[pl.ds(off, CHUNK)], t_v, st[b]).wait()
        pltpu.make_async_copy(
            depths_hbm.at[pl.ds(off, CHUNK)], d_v, sd[b]).wait()

        @pl.when(ci + 1 < NCH)
        def _prefetch():
            noff = base + (ci + 1) * CHUNK
            pltpu.async_copy(
                times_hbm.at[pl.ds(noff, CHUNK)], t_bufs[1 - b], st[1 - b])
            pltpu.async_copy(
                depths_hbm.at[pl.ds(noff, CHUNK)], d_bufs[1 - b], sd[1 - b])

        @pl.when(ci >= 2)
        def _drain_prev_out():
            pltpu.make_async_copy(
                o_v, out_hbm.at[pl.ds(off, CHUNK)], so[b]).wait()

        def do_group(i):
            sl = pl.ds(i * 16, 16)
            t = t_v[sl]
            d = d_v[sl]
            bins = (t * np.float32(1.0 / BIN)).astype(jnp.int32)
            j = (d * np.float32(D - 1)).astype(jnp.int32)
            q = jnp.minimum(jnp.maximum(j >> 1, jnp.int32(1)), jnp.int32(6))
            p0 = q - jnp.int32(1)
            idx0 = p0 * jnp.int32(NUM_T) + bins
            t0 = d - p0.astype(jnp.float32) * np.float32(2.0 / (D - 1))
            num = jnp.zeros((16,), jnp.float32)
            den = jnp.zeros((16,), jnp.float32)
            for k in range(3):
                x = plsc.load_gather(tab_v, [idx0 + jnp.int32(k * NUM_T)])
                ve = lax.bitcast_convert_type(x << 16, jnp.float32)
                vo = lax.bitcast_convert_type(
                    x & jnp.int32(-65536), jnp.float32)
                te = t0 - np.float32(k * 2.0 / (D - 1))
                to = te - np.float32(1.0 / (D - 1))
                ce = jnp.maximum(np.float32(DS) - jnp.abs(te), np.float32(0.0))
                co = jnp.maximum(np.float32(DS) - jnp.abs(to), np.float32(0.0))
                num = num + ce * ve + co * vo
                den = den + ce + co
            x = np.float32(EPS2) + den
            r = np.float32(_R0) * (np.float32(2.0) - x * np.float32(_R0))
            o_v[sl] = num * r

        def vec_body(i, c2):
            do_group(i * 4)
            do_group(i * 4 + 1)
            do_group(i * 4 + 2)
            do_group(i * 4 + 3)
            return c2

        lax.fori_loop(0, CHUNK // 64, vec_body, 0)
        pltpu.async_copy(o_v, out_hbm.at[pl.ds(off, CHUNK)], so[b])

      return chunk_body

    chunk_bodies = (make_chunk_body(0), make_chunk_body(1))

    def pair_body(g, carry):
        chunk_bodies[0](g * 2, ta_v, da_v, oa_v)
        chunk_bodies[1](g * 2 + 1, tb_v, db_v, ob_v)
        return carry

    lax.fori_loop(0, NCH // 2, pair_body, 0)
    for b in range(2):
        pltpu.make_async_copy(
            o_bufs[b], out_hbm.at[pl.ds(base, CHUNK)], so[b]).wait()


@functools.cache
def _sc_predict():
    return pl.kernel(
        _sc_body,
        out_type=jax.ShapeDtypeStruct((N,), jnp.float32),
        mesh=plsc.VectorSubcoreMesh(core_axis_name="c", subcore_axis_name="s",
                                    num_cores=NUM_CORES,
                                    num_subcores=NUM_SUBCORES),
        compiler_params=pltpu.CompilerParams(needs_layout_passes=False),
        scratch_types=(
            [pltpu.VMEM((NPAIR * NUM_T,), jnp.int32)]
            + [pltpu.VMEM((CHUNK,), jnp.float32) for _ in range(6)]
            + [pltpu.SemaphoreType.DMA for _ in range(6)]
        ),
    )


def kernel(times, depths, motion):
    table = _build_table(motion)
    return _sc_predict()(table, times, depths)
